# baseline (speedup 1.0000x reference)
import functools

import jax
import jax.numpy as jnp
from jax import lax
from jax.experimental import pallas as pl
from jax.experimental.pallas import tpu as pltpu
from jax.experimental.pallas import tpu_sc as plsc

VOCAB = 1000
HIDDEN = 128
B = 16384
L = 200

_NC = 2
_NS = 16
_NW = _NC * _NS
_R = B // _NW
_NCHUNK = 4
_CC = _R // _NCHUNK
_VPAD = 1024


def _prep_body(wf_ref, win_t_ref, wc1_t_ref, wc2_ref, u_ref, vw_ref):
    wf_text = wf_ref[0:1, 0:HIDDEN]
    u = lax.dot_general(wf_text, win_t_ref[...],
                        (((1,), (1,)), ((), ())),
                        preferred_element_type=jnp.float32,
                        precision=lax.Precision.HIGHEST)
    u = jnp.concatenate([u, jnp.zeros((1, _VPAD - VOCAB), jnp.float32)],
                        axis=1)
    col = lax.broadcasted_iota(jnp.int32, (1, _VPAD), 1)
    u_ref[...] = jnp.where(col == 1, 0.0, u)
    v = lax.dot_general(wc2_ref[...], wc1_t_ref[...],
                        (((1,), (1,)), ((), ())),
                        preferred_element_type=jnp.float32,
                        precision=lax.Precision.HIGHEST)
    v = jnp.concatenate([v, jnp.zeros((1, HIDDEN - 10), jnp.float32)],
                        axis=1)
    colv = lax.broadcasted_iota(jnp.int32, (1, HIDDEN), 1)
    vm = jnp.where(colv == 1, 0.0, v)
    w2 = vm * wf_ref[0:1, HIDDEN:HIDDEN + 1]
    vw_ref[...] = jnp.concatenate([vm, w2], axis=0)


_prep = pl.pallas_call(
    _prep_body,
    out_shape=(
        jax.ShapeDtypeStruct((1, _VPAD), jnp.float32),
        jax.ShapeDtypeStruct((2, HIDDEN), jnp.float32),
    ),
)


def _sc_main_body(ids_hbm, cat_hbm, y_hbm, u_hbm, vw_hbm,
                  conf_hbm, fin_hbm, loss_hbm,
                  ids_a, ids_b, u_v, vw_v, cat_v, y_v, t_v, conf_v, fin_v,
                  loss_v, sem_a, sem_b):
    wid = lax.axis_index("s") * _NC + lax.axis_index("c")
    base = wid * _R

    bufs = [(ids_a, sem_a), (ids_b, sem_b)]
    copies = [None] * _NCHUNK
    copies[0] = pltpu.async_copy(ids_hbm.at[:, pl.ds(base, _CC)], ids_a, sem_a)
    pltpu.sync_copy(u_hbm, u_v)
    pltpu.sync_copy(vw_hbm, vw_v)
    pltpu.sync_copy(cat_hbm.at[pl.ds(base, _R)], cat_v)
    pltpu.sync_copy(y_hbm.at[pl.ds(base, _R)], y_v)

    zero16 = jnp.zeros((16,), jnp.float32)

    for chunk in range(_NCHUNK):
        ids_v, _ = bufs[chunk % 2]
        copies[chunk].wait()
        if chunk + 1 < _NCHUNK:
            nbuf, nsem = bufs[(chunk + 1) % 2]
            copies[chunk + 1] = pltpu.async_copy(
                ids_hbm.at[:, pl.ds(base + (chunk + 1) * _CC, _CC)],
                nbuf, nsem)

        for g in range(_CC // 16):
            col0 = g * 16

            def tok_body(t, accs, ids_v=ids_v, col0=col0):
                t4 = t * 4
                g0 = plsc.load_gather(u_v, [ids_v[t4, pl.ds(col0, 16)]])
                g1 = plsc.load_gather(u_v, [ids_v[t4 + 1, pl.ds(col0, 16)]])
                g2 = plsc.load_gather(u_v, [ids_v[t4 + 2, pl.ds(col0, 16)]])
                g3 = plsc.load_gather(u_v, [ids_v[t4 + 3, pl.ds(col0, 16)]])
                return (accs[0] + g0, accs[1] + g1,
                        accs[2] + g2, accs[3] + g3)

            a = lax.fori_loop(0, L // 4, tok_body,
                              (zero16, zero16, zero16, zero16), unroll=5)
            t_v[pl.ds(chunk * _CC + col0, 16)] = (a[0] + a[1]) + (a[2] + a[3])

    def ep_body(g, carry):
        lc, lf = carry
        t16 = t_v[pl.ds(g * 16, 16)]
        c16 = cat_v[pl.ds(g * 16, 16)]
        y16 = y_v[pl.ds(g * 16, 16)]
        cpv = plsc.load_gather(vw_v, [c16])
        w2v = plsc.load_gather(vw_v, [c16 + HIDDEN])
        fpv = t16 + w2v
        conf_v[pl.ds(g * 16, 16)] = cpv
        fin_v[pl.ds(g * 16, 16)] = fpv
        dc = cpv - y16
        df = fpv - y16
        return (lc + dc * dc, lf + df * df)

    lc, lf = lax.fori_loop(0, _R // 16, ep_body, (zero16, zero16),
                           unroll=False)
    loss_v[pl.ds(0, 16)] = lc
    loss_v[pl.ds(16, 16)] = lf

    pltpu.sync_copy(conf_v, conf_hbm.at[pl.ds(base, _R)])
    pltpu.sync_copy(fin_v, fin_hbm.at[pl.ds(base, _R)])
    pltpu.sync_copy(loss_v, loss_hbm.at[pl.ds(wid * 32, 32)])


@functools.cache
def _get_sc_main():
    mesh = plsc.VectorSubcoreMesh(core_axis_name="c", subcore_axis_name="s",
                                  num_cores=_NC, num_subcores=_NS)
    return pl.kernel(
        _sc_main_body,
        mesh=mesh,
        compiler_params=pltpu.CompilerParams(needs_layout_passes=False),
        out_type=(
            jax.ShapeDtypeStruct((B,), jnp.float32),
            jax.ShapeDtypeStruct((B,), jnp.float32),
            jax.ShapeDtypeStruct((_NW * 32,), jnp.float32),
        ),
        scratch_types=[
            pltpu.VMEM((L, _CC), jnp.int32),
            pltpu.VMEM((L, _CC), jnp.int32),
            pltpu.VMEM((_VPAD,), jnp.float32),
            pltpu.VMEM((2 * HIDDEN,), jnp.float32),
            pltpu.VMEM((_R,), jnp.int32),
            pltpu.VMEM((_R,), jnp.float32),
            pltpu.VMEM((_R,), jnp.float32),
            pltpu.VMEM((_R,), jnp.float32),
            pltpu.VMEM((_R,), jnp.float32),
            pltpu.VMEM((32,), jnp.float32),
            pltpu.SemaphoreType.DMA,
            pltpu.SemaphoreType.DMA,
        ],
    )


def kernel(input, confound_cat, outcome_y, W_in, Wc1, Wc2, Wf):
    u2d, vw2d = _prep(Wf, W_in.T, Wc1.T, Wc2)
    u_flat = u2d.reshape(_VPAD)
    vw_flat = vw2d.reshape(2 * HIDDEN)

    conf, fin, loss = _get_sc_main()(input.T, confound_cat, outcome_y,
                                     u_flat, vw_flat)

    confound_pred = conf[:, None]
    final_pred = fin[:, None]
    loss2 = loss.reshape(_NW, 32)
    confound_loss = jnp.sum(loss2[:, :16]) / B
    final_loss = jnp.sum(loss2[:, 16:]) / B
    return (confound_pred, confound_loss, final_pred, final_loss)


# speedup vs baseline: 1.0032x; 1.0032x over previous
import functools

import jax
import jax.numpy as jnp
from jax import lax
from jax.experimental import pallas as pl
from jax.experimental.pallas import tpu as pltpu
from jax.experimental.pallas import tpu_sc as plsc

VOCAB = 1000
HIDDEN = 128
B = 16384
L = 200

_NC = 2
_NS = 16
_NW = _NC * _NS
_R = B // _NW
_NCHUNK = 4
_CC = _R // _NCHUNK
_VPAD = 1024


def _prep_body(wf_ref, win_t_ref, wc1_t_ref, wc2_ref, u_ref, vw_ref):
    wf_text = wf_ref[0:1, 0:HIDDEN]
    u = lax.dot_general(wf_text, win_t_ref[...],
                        (((1,), (1,)), ((), ())),
                        preferred_element_type=jnp.float32,
                        precision=lax.Precision.HIGHEST)
    u = jnp.concatenate([u, jnp.zeros((1, _VPAD - VOCAB), jnp.float32)],
                        axis=1)
    col = lax.broadcasted_iota(jnp.int32, (1, _VPAD), 1)
    u_ref[...] = jnp.where(col == 1, 0.0, u)
    v = lax.dot_general(wc2_ref[...], wc1_t_ref[...],
                        (((1,), (1,)), ((), ())),
                        preferred_element_type=jnp.float32,
                        precision=lax.Precision.HIGHEST)
    v = jnp.concatenate([v, jnp.zeros((1, HIDDEN - 10), jnp.float32)],
                        axis=1)
    colv = lax.broadcasted_iota(jnp.int32, (1, HIDDEN), 1)
    vm = jnp.where(colv == 1, 0.0, v)
    w2 = vm * wf_ref[0:1, HIDDEN:HIDDEN + 1]
    vw_ref[...] = jnp.concatenate([vm, w2], axis=0)


_prep = pl.pallas_call(
    _prep_body,
    out_shape=(
        jax.ShapeDtypeStruct((1, _VPAD), jnp.float32),
        jax.ShapeDtypeStruct((2, HIDDEN), jnp.float32),
    ),
)


def _sc_main_body(ids_hbm, cat_hbm, y_hbm, u_hbm, vw_hbm,
                  conf_hbm, fin_hbm, loss_hbm,
                  ids_a, ids_b, u_v, vw_v, cat_v, y_v, t_v, conf_v, fin_v,
                  loss_v, sem_a, sem_b):
    wid = lax.axis_index("s") * _NC + lax.axis_index("c")
    base = wid * _R

    bufs = [(ids_a, sem_a), (ids_b, sem_b)]
    copies = [None] * _NCHUNK
    copies[0] = pltpu.async_copy(ids_hbm.at[:, pl.ds(base, _CC)], ids_a, sem_a)
    pltpu.sync_copy(u_hbm, u_v)
    pltpu.sync_copy(vw_hbm, vw_v)
    pltpu.sync_copy(cat_hbm.at[pl.ds(base, _R)], cat_v)
    pltpu.sync_copy(y_hbm.at[pl.ds(base, _R)], y_v)

    zero16 = jnp.zeros((16,), jnp.float32)

    for chunk in range(_NCHUNK):
        ids_v, _ = bufs[chunk % 2]
        copies[chunk].wait()
        if chunk + 1 < _NCHUNK:
            nbuf, nsem = bufs[(chunk + 1) % 2]
            copies[chunk + 1] = pltpu.async_copy(
                ids_hbm.at[:, pl.ds(base + (chunk + 1) * _CC, _CC)],
                nbuf, nsem)

        for g in range(_CC // 16):
            col0 = g * 16

            def tok_body(t, accs, ids_v=ids_v, col0=col0):
                t8 = t * 8
                gs = [plsc.load_gather(u_v, [ids_v[t8 + j, pl.ds(col0, 16)]])
                      for j in range(8)]
                return tuple(a + g for a, g in zip(accs, gs))

            a = lax.fori_loop(0, L // 8, tok_body, (zero16,) * 8, unroll=2)
            t_v[pl.ds(chunk * _CC + col0, 16)] = (
                ((a[0] + a[1]) + (a[2] + a[3]))
                + ((a[4] + a[5]) + (a[6] + a[7])))

    def ep_body(g, carry):
        lc, lf = carry
        t16 = t_v[pl.ds(g * 16, 16)]
        c16 = cat_v[pl.ds(g * 16, 16)]
        y16 = y_v[pl.ds(g * 16, 16)]
        cpv = plsc.load_gather(vw_v, [c16])
        w2v = plsc.load_gather(vw_v, [c16 + HIDDEN])
        fpv = t16 + w2v
        conf_v[pl.ds(g * 16, 16)] = cpv
        fin_v[pl.ds(g * 16, 16)] = fpv
        dc = cpv - y16
        df = fpv - y16
        return (lc + dc * dc, lf + df * df)

    lc, lf = lax.fori_loop(0, _R // 16, ep_body, (zero16, zero16),
                           unroll=False)
    loss_v[pl.ds(0, 16)] = lc
    loss_v[pl.ds(16, 16)] = lf

    pltpu.sync_copy(conf_v, conf_hbm.at[pl.ds(base, _R)])
    pltpu.sync_copy(fin_v, fin_hbm.at[pl.ds(base, _R)])
    pltpu.sync_copy(loss_v, loss_hbm.at[pl.ds(wid * 32, 32)])


@functools.cache
def _get_sc_main():
    mesh = plsc.VectorSubcoreMesh(core_axis_name="c", subcore_axis_name="s",
                                  num_cores=_NC, num_subcores=_NS)
    return pl.kernel(
        _sc_main_body,
        mesh=mesh,
        compiler_params=pltpu.CompilerParams(needs_layout_passes=False),
        out_type=(
            jax.ShapeDtypeStruct((B,), jnp.float32),
            jax.ShapeDtypeStruct((B,), jnp.float32),
            jax.ShapeDtypeStruct((_NW * 32,), jnp.float32),
        ),
        scratch_types=[
            pltpu.VMEM((L, _CC), jnp.int32),
            pltpu.VMEM((L, _CC), jnp.int32),
            pltpu.VMEM((_VPAD,), jnp.float32),
            pltpu.VMEM((2 * HIDDEN,), jnp.float32),
            pltpu.VMEM((_R,), jnp.int32),
            pltpu.VMEM((_R,), jnp.float32),
            pltpu.VMEM((_R,), jnp.float32),
            pltpu.VMEM((_R,), jnp.float32),
            pltpu.VMEM((_R,), jnp.float32),
            pltpu.VMEM((32,), jnp.float32),
            pltpu.SemaphoreType.DMA,
            pltpu.SemaphoreType.DMA,
        ],
    )


def kernel(input, confound_cat, outcome_y, W_in, Wc1, Wc2, Wf):
    u2d, vw2d = _prep(Wf, W_in.T, Wc1.T, Wc2)
    u_flat = u2d.reshape(_VPAD)
    vw_flat = vw2d.reshape(2 * HIDDEN)

    conf, fin, loss = _get_sc_main()(input.T, confound_cat, outcome_y,
                                     u_flat, vw_flat)

    confound_pred = conf[:, None]
    final_pred = fin[:, None]
    loss2 = loss.reshape(_NW, 32)
    confound_loss = jnp.sum(loss2[:, :16]) / B
    final_loss = jnp.sum(loss2[:, 16:]) / B
    return (confound_pred, confound_loss, final_pred, final_loss)


# speedup vs baseline: 1.0632x; 1.0598x over previous
import functools

import jax
import jax.numpy as jnp
from jax import lax
from jax.experimental import pallas as pl
from jax.experimental.pallas import tpu as pltpu
from jax.experimental.pallas import tpu_sc as plsc

VOCAB = 1000
HIDDEN = 128
B = 16384
L = 200

_NC = 2
_NS = 16
_NW = _NC * _NS
_R = B // _NW
_NCHUNK = 4
_CC = _R // _NCHUNK
_VPAD = 1024


def _prep_body(wf_ref, win_t_ref, wc1_t_ref, wc2_ref, u_ref, vw_ref):
    wf_text = wf_ref[0:1, 0:HIDDEN]
    u = lax.dot_general(wf_text, win_t_ref[...],
                        (((1,), (1,)), ((), ())),
                        preferred_element_type=jnp.float32,
                        precision=lax.Precision.HIGHEST)
    u = jnp.concatenate([u, jnp.zeros((1, _VPAD - VOCAB), jnp.float32)],
                        axis=1)
    col = lax.broadcasted_iota(jnp.int32, (1, _VPAD), 1)
    u_ref[...] = jnp.where(col == 1, 0.0, u)
    v = lax.dot_general(wc2_ref[...], wc1_t_ref[...],
                        (((1,), (1,)), ((), ())),
                        preferred_element_type=jnp.float32,
                        precision=lax.Precision.HIGHEST)
    v = jnp.concatenate([v, jnp.zeros((1, HIDDEN - 10), jnp.float32)],
                        axis=1)
    colv = lax.broadcasted_iota(jnp.int32, (1, HIDDEN), 1)
    vm = jnp.where(colv == 1, 0.0, v)
    w2 = vm * wf_ref[0:1, HIDDEN:HIDDEN + 1]
    vw_ref[...] = jnp.concatenate([vm, w2], axis=0)


_prep = pl.pallas_call(
    _prep_body,
    out_shape=(
        jax.ShapeDtypeStruct((1, _VPAD), jnp.float32),
        jax.ShapeDtypeStruct((2, HIDDEN), jnp.float32),
    ),
)


def _sc_main_body(ids_hbm, cat_hbm, y_hbm, u_hbm, vw_hbm,
                  conf_hbm, fin_hbm, loss_hbm,
                  ids_a, ids_b, u_v, vw_v, cat_v, y_v, t_v, conf_v, fin_v,
                  loss_v, sem_a, sem_b):
    wid = lax.axis_index("s") * _NC + lax.axis_index("c")
    base = wid * _R

    bufs = [(ids_a, sem_a), (ids_b, sem_b)]
    copies = [None] * _NCHUNK
    copies[0] = pltpu.async_copy(ids_hbm.at[:, pl.ds(base, _CC)], ids_a, sem_a)
    pltpu.sync_copy(u_hbm, u_v)
    pltpu.sync_copy(vw_hbm, vw_v)
    pltpu.sync_copy(cat_hbm.at[pl.ds(base, _R)], cat_v)
    pltpu.sync_copy(y_hbm.at[pl.ds(base, _R)], y_v)

    zero16 = jnp.zeros((16,), jnp.float32)

    for chunk in range(_NCHUNK):
        ids_v, _ = bufs[chunk % 2]
        copies[chunk].wait()
        if chunk + 1 < _NCHUNK:
            nbuf, nsem = bufs[(chunk + 1) % 2]
            copies[chunk + 1] = pltpu.async_copy(
                ids_hbm.at[:, pl.ds(base + (chunk + 1) * _CC, _CC)],
                nbuf, nsem)

        for g in range(_CC // 16):
            col0 = g * 16

            def tok_body(t, accs, ids_v=ids_v, col0=col0):
                t4 = t * 4
                g0 = plsc.load_gather(u_v, [ids_v[t4, pl.ds(col0, 16)]])
                g1 = plsc.load_gather(u_v, [ids_v[t4 + 1, pl.ds(col0, 16)]])
                g2 = plsc.load_gather(u_v, [ids_v[t4 + 2, pl.ds(col0, 16)]])
                g3 = plsc.load_gather(u_v, [ids_v[t4 + 3, pl.ds(col0, 16)]])
                return (accs[0] + g0, accs[1] + g1,
                        accs[2] + g2, accs[3] + g3)

            a = lax.fori_loop(0, L // 4, tok_body,
                              (zero16, zero16, zero16, zero16), unroll=1)
            t_v[pl.ds(chunk * _CC + col0, 16)] = (a[0] + a[1]) + (a[2] + a[3])

    def ep_body(g, carry):
        lc, lf = carry
        t16 = t_v[pl.ds(g * 16, 16)]
        c16 = cat_v[pl.ds(g * 16, 16)]
        y16 = y_v[pl.ds(g * 16, 16)]
        cpv = plsc.load_gather(vw_v, [c16])
        w2v = plsc.load_gather(vw_v, [c16 + HIDDEN])
        fpv = t16 + w2v
        conf_v[pl.ds(g * 16, 16)] = cpv
        fin_v[pl.ds(g * 16, 16)] = fpv
        dc = cpv - y16
        df = fpv - y16
        return (lc + dc * dc, lf + df * df)

    lc, lf = lax.fori_loop(0, _R // 16, ep_body, (zero16, zero16),
                           unroll=False)
    loss_v[pl.ds(0, 16)] = lc
    loss_v[pl.ds(16, 16)] = lf

    pltpu.sync_copy(conf_v, conf_hbm.at[pl.ds(base, _R)])
    pltpu.sync_copy(fin_v, fin_hbm.at[pl.ds(base, _R)])
    pltpu.sync_copy(loss_v, loss_hbm.at[pl.ds(wid * 32, 32)])


@functools.cache
def _get_sc_main():
    mesh = plsc.VectorSubcoreMesh(core_axis_name="c", subcore_axis_name="s",
                                  num_cores=_NC, num_subcores=_NS)
    return pl.kernel(
        _sc_main_body,
        mesh=mesh,
        compiler_params=pltpu.CompilerParams(needs_layout_passes=False),
        out_type=(
            jax.ShapeDtypeStruct((B,), jnp.float32),
            jax.ShapeDtypeStruct((B,), jnp.float32),
            jax.ShapeDtypeStruct((_NW * 32,), jnp.float32),
        ),
        scratch_types=[
            pltpu.VMEM((L, _CC), jnp.int32),
            pltpu.VMEM((L, _CC), jnp.int32),
            pltpu.VMEM((_VPAD,), jnp.float32),
            pltpu.VMEM((2 * HIDDEN,), jnp.float32),
            pltpu.VMEM((_R,), jnp.int32),
            pltpu.VMEM((_R,), jnp.float32),
            pltpu.VMEM((_R,), jnp.float32),
            pltpu.VMEM((_R,), jnp.float32),
            pltpu.VMEM((_R,), jnp.float32),
            pltpu.VMEM((32,), jnp.float32),
            pltpu.SemaphoreType.DMA,
            pltpu.SemaphoreType.DMA,
        ],
    )


def kernel(input, confound_cat, outcome_y, W_in, Wc1, Wc2, Wf):
    u2d, vw2d = _prep(Wf, W_in.T, Wc1.T, Wc2)
    u_flat = u2d.reshape(_VPAD)
    vw_flat = vw2d.reshape(2 * HIDDEN)

    conf, fin, loss = _get_sc_main()(input.T, confound_cat, outcome_y,
                                     u_flat, vw_flat)

    confound_pred = conf[:, None]
    final_pred = fin[:, None]
    loss2 = loss.reshape(_NW, 32)
    confound_loss = jnp.sum(loss2[:, :16]) / B
    final_loss = jnp.sum(loss2[:, 16:]) / B
    return (confound_pred, confound_loss, final_pred, final_loss)


# speedup vs baseline: 1.0853x; 1.0208x over previous
import functools

import jax
import jax.numpy as jnp
from jax import lax
from jax.experimental import pallas as pl
from jax.experimental.pallas import tpu as pltpu
from jax.experimental.pallas import tpu_sc as plsc

VOCAB = 1000
HIDDEN = 128
B = 16384
L = 200

_NC = 2
_NS = 16
_NW = _NC * _NS
_R = B // _NW
_NCHUNK = 4
_CC = _R // _NCHUNK
_VPAD = 1024


def _prep_body(wf_ref, win_t_ref, wc1_t_ref, wc2_ref, u_ref, vw_ref):
    wf_text = wf_ref[0:1, 0:HIDDEN]
    u = lax.dot_general(wf_text, win_t_ref[...],
                        (((1,), (1,)), ((), ())),
                        preferred_element_type=jnp.float32,
                        precision=lax.Precision.HIGHEST)
    u = jnp.concatenate([u, jnp.zeros((1, _VPAD - VOCAB), jnp.float32)],
                        axis=1)
    col = lax.broadcasted_iota(jnp.int32, (1, _VPAD), 1)
    u_ref[...] = jnp.where(col == 1, 0.0, u)
    v = lax.dot_general(wc2_ref[...], wc1_t_ref[...],
                        (((1,), (1,)), ((), ())),
                        preferred_element_type=jnp.float32,
                        precision=lax.Precision.HIGHEST)
    v = jnp.concatenate([v, jnp.zeros((1, HIDDEN - 10), jnp.float32)],
                        axis=1)
    colv = lax.broadcasted_iota(jnp.int32, (1, HIDDEN), 1)
    vm = jnp.where(colv == 1, 0.0, v)
    w2 = vm * wf_ref[0:1, HIDDEN:HIDDEN + 1]
    vw_ref[...] = jnp.concatenate([vm, w2], axis=0)


_prep = pl.pallas_call(
    _prep_body,
    out_shape=(
        jax.ShapeDtypeStruct((1, _VPAD), jnp.float32),
        jax.ShapeDtypeStruct((2, HIDDEN), jnp.float32),
    ),
)


def _sc_main_body(ids_hbm, cat_hbm, y_hbm, u_hbm, vw_hbm,
                  conf_hbm, fin_hbm, loss_hbm,
                  ids_a, ids_b, u_v, vw_v, cat_v, y_v, t_v, conf_v, fin_v,
                  loss_v, sem_a, sem_b):
    wid = lax.axis_index("s") * _NC + lax.axis_index("c")
    base = wid * _R

    bufs = [(ids_a, sem_a), (ids_b, sem_b)]
    copies = [None] * _NCHUNK
    copies[0] = pltpu.async_copy(ids_hbm.at[:, pl.ds(base, _CC)], ids_a, sem_a)
    pltpu.sync_copy(u_hbm, u_v)
    pltpu.sync_copy(vw_hbm, vw_v)
    pltpu.sync_copy(cat_hbm.at[pl.ds(base, _R)], cat_v)
    pltpu.sync_copy(y_hbm.at[pl.ds(base, _R)], y_v)

    zero16 = jnp.zeros((16,), jnp.float32)

    for chunk in range(_NCHUNK):
        ids_v, _ = bufs[chunk % 2]
        copies[chunk].wait()
        if chunk + 1 < _NCHUNK:
            nbuf, nsem = bufs[(chunk + 1) % 2]
            copies[chunk + 1] = pltpu.async_copy(
                ids_hbm.at[:, pl.ds(base + (chunk + 1) * _CC, _CC)],
                nbuf, nsem)

        for g in range(_CC // 16):
            col0 = g * 16

            def tok_body(t, accs, ids_v=ids_v, col0=col0):
                t4 = t * 4
                g0 = plsc.load_gather(u_v, [ids_v[t4, pl.ds(col0, 16)]])
                g1 = plsc.load_gather(u_v, [ids_v[t4 + 1, pl.ds(col0, 16)]])
                g2 = plsc.load_gather(u_v, [ids_v[t4 + 2, pl.ds(col0, 16)]])
                g3 = plsc.load_gather(u_v, [ids_v[t4 + 3, pl.ds(col0, 16)]])
                return (accs[0] + g0, accs[1] + g1,
                        accs[2] + g2, accs[3] + g3)

            a = lax.fori_loop(0, L // 4, tok_body,
                              (zero16, zero16, zero16, zero16), unroll=2)
            t_v[pl.ds(chunk * _CC + col0, 16)] = (a[0] + a[1]) + (a[2] + a[3])

    def ep_body(g, carry):
        lc, lf = carry
        t16 = t_v[pl.ds(g * 16, 16)]
        c16 = cat_v[pl.ds(g * 16, 16)]
        y16 = y_v[pl.ds(g * 16, 16)]
        cpv = plsc.load_gather(vw_v, [c16])
        w2v = plsc.load_gather(vw_v, [c16 + HIDDEN])
        fpv = t16 + w2v
        conf_v[pl.ds(g * 16, 16)] = cpv
        fin_v[pl.ds(g * 16, 16)] = fpv
        dc = cpv - y16
        df = fpv - y16
        return (lc + dc * dc, lf + df * df)

    lc, lf = lax.fori_loop(0, _R // 16, ep_body, (zero16, zero16),
                           unroll=False)
    loss_v[pl.ds(0, 16)] = lc
    loss_v[pl.ds(16, 16)] = lf

    pltpu.sync_copy(conf_v, conf_hbm.at[pl.ds(base, _R)])
    pltpu.sync_copy(fin_v, fin_hbm.at[pl.ds(base, _R)])
    pltpu.sync_copy(loss_v, loss_hbm.at[pl.ds(wid * 32, 32)])


@functools.cache
def _get_sc_main():
    mesh = plsc.VectorSubcoreMesh(core_axis_name="c", subcore_axis_name="s",
                                  num_cores=_NC, num_subcores=_NS)
    return pl.kernel(
        _sc_main_body,
        mesh=mesh,
        compiler_params=pltpu.CompilerParams(needs_layout_passes=False),
        out_type=(
            jax.ShapeDtypeStruct((B,), jnp.float32),
            jax.ShapeDtypeStruct((B,), jnp.float32),
            jax.ShapeDtypeStruct((_NW * 32,), jnp.float32),
        ),
        scratch_types=[
            pltpu.VMEM((L, _CC), jnp.int32),
            pltpu.VMEM((L, _CC), jnp.int32),
            pltpu.VMEM((_VPAD,), jnp.float32),
            pltpu.VMEM((2 * HIDDEN,), jnp.float32),
            pltpu.VMEM((_R,), jnp.int32),
            pltpu.VMEM((_R,), jnp.float32),
            pltpu.VMEM((_R,), jnp.float32),
            pltpu.VMEM((_R,), jnp.float32),
            pltpu.VMEM((_R,), jnp.float32),
            pltpu.VMEM((32,), jnp.float32),
            pltpu.SemaphoreType.DMA,
            pltpu.SemaphoreType.DMA,
        ],
    )


def kernel(input, confound_cat, outcome_y, W_in, Wc1, Wc2, Wf):
    u2d, vw2d = _prep(Wf, W_in.T, Wc1.T, Wc2)
    u_flat = u2d.reshape(_VPAD)
    vw_flat = vw2d.reshape(2 * HIDDEN)

    conf, fin, loss = _get_sc_main()(input.T, confound_cat, outcome_y,
                                     u_flat, vw_flat)

    confound_pred = conf[:, None]
    final_pred = fin[:, None]
    loss2 = loss.reshape(_NW, 32)
    confound_loss = jnp.sum(loss2[:, :16]) / B
    final_loss = jnp.sum(loss2[:, 16:]) / B
    return (confound_pred, confound_loss, final_pred, final_loss)
